# Initial kernel scaffold; baseline (speedup 1.0000x reference)
import jax
import jax.numpy as jnp
from jax.experimental import pallas as pl


def kernel(coords_grid, anchor_P):
    raise NotImplementedError("write your pallas kernel here")


# speedup vs baseline: 9.3174x; 9.3174x over previous
import jax
import jax.numpy as jnp
from jax.experimental import pallas as pl

M = 16
TOPK = 4
HB = 32


def _nms_body(pj_ref, out_ref):
    x = []
    y = []
    for m in range(M):
        px = pj_ref[0, m, 0]
        py = pj_ref[0, m, 1]
        pz = pj_ref[0, m, 2]
        x.append(px / pz)
        y.append(py / pz)

    shape = x[0].shape
    ones = jnp.ones(shape, dtype=jnp.bool_)

    supp = [None] * M
    keep = [None] * M
    for m in range(M):
        km = ones if supp[m] is None else jnp.logical_not(supp[m])
        keep[m] = km
        for j in range(m + 1, M):
            dx = x[j] - x[m]
            dy = y[j] - y[m]
            d = jnp.sqrt(dx * dx + dy * dy)
            c = jnp.logical_and(km, d <= 2.0)
            supp[j] = c if supp[j] is None else jnp.logical_or(supp[j], c)

    zero = jnp.zeros(shape, dtype=jnp.int32)
    cnt = zero
    kept_before = [None] * M
    for m in range(M):
        kept_before[m] = cnt
        cnt = cnt + keep[m].astype(jnp.int32)
    for k in range(TOPK):
        acc = zero
        for m in range(M):
            rank_m = jnp.where(keep[m], kept_before[m],
                               cnt + (m - kept_before[m]))
            acc = acc | jnp.where(rank_m == k, jnp.int32(m), 0)
        out_ref[0, k] = acc


@jax.jit
def kernel(coords_grid, anchor_P):
    N, M_, _, H, W = coords_grid.shape
    cg = jnp.transpose(coords_grid, (0, 2, 3, 4, 1)).reshape(N, 3, H * W, M_)
    proj = jnp.einsum('nij,njkm->nikm', anchor_P, cg)
    pj = jnp.transpose(proj, (0, 3, 1, 2)).reshape(N, M_, 3, H, W)
    grid = (N, H // HB)
    out = pl.pallas_call(
        _nms_body,
        grid=grid,
        in_specs=[
            pl.BlockSpec((1, M_, 3, HB, W), lambda n, h: (n, 0, 0, h, 0)),
        ],
        out_specs=pl.BlockSpec((1, TOPK, HB, W), lambda n, h: (n, 0, h, 0)),
        out_shape=jax.ShapeDtypeStruct((N, TOPK, H, W), jnp.int32),
    )(pj)
    return jnp.transpose(out, (0, 2, 3, 1)).astype(jnp.int64)


# speedup vs baseline: 11.1860x; 1.2005x over previous
import jax
import jax.numpy as jnp
from jax.experimental import pallas as pl

M = 16
TOPK = 4
HB = 32


def _nms_body(pj_ref, out_ref):
    x = []
    y = []
    for m in range(M):
        px = pj_ref[0, m, 0]
        py = pj_ref[0, m, 1]
        pz = pj_ref[0, m, 2]
        x.append(px / pz)
        y.append(py / pz)

    shape = x[0].shape
    ones = jnp.ones(shape, dtype=jnp.bool_)

    supp = [None] * M
    keep = [None] * M
    for m in range(M):
        km = ones if supp[m] is None else jnp.logical_not(supp[m])
        keep[m] = km
        for j in range(m + 1, M):
            dx = x[j] - x[m]
            dy = y[j] - y[m]
            d = jnp.sqrt(dx * dx + dy * dy)
            c = jnp.logical_and(km, d <= 2.0)
            supp[j] = c if supp[j] is None else jnp.logical_or(supp[j], c)

    zero = jnp.zeros(shape, dtype=jnp.int32)
    cnt = zero
    kept_before = [None] * M
    for m in range(M):
        kept_before[m] = cnt
        cnt = cnt + keep[m].astype(jnp.int32)
    for k in range(TOPK):
        acc = zero
        for m in range(M):
            rank_m = jnp.where(keep[m], kept_before[m],
                               cnt + (m - kept_before[m]))
            acc = acc | jnp.where(rank_m == k, jnp.int32(m), 0)
        out_ref[0, k] = acc


@jax.jit
def kernel(coords_grid, anchor_P):
    N, M_, _, H, W = coords_grid.shape
    pj = jnp.einsum('nij,nmjhw->nmihw', anchor_P, coords_grid)
    grid = (N, H // HB)
    out = pl.pallas_call(
        _nms_body,
        grid=grid,
        in_specs=[
            pl.BlockSpec((1, M_, 3, HB, W), lambda n, h: (n, 0, 0, h, 0)),
        ],
        out_specs=pl.BlockSpec((1, TOPK, HB, W), lambda n, h: (n, 0, h, 0)),
        out_shape=jax.ShapeDtypeStruct((N, TOPK, H, W), jnp.int32),
    )(pj)
    return jnp.transpose(out, (0, 2, 3, 1)).astype(jnp.int64)
